# baseline (speedup 1.0000x reference)
import functools

import jax
import jax.numpy as jnp
from jax import lax
from jax.experimental import pallas as pl
from jax.experimental.pallas import tpu as pltpu
from jax.experimental.pallas import tpu_sc as plsc

_L = 16
_G = 128
_XB = 32


def _build(B, H, V, D, NW):
    n_groups = (B // _G) * H
    gpw = n_groups // NW
    n_pairs = gpw // 2
    DT, DS = D // 8, 8

    mesh = plsc.VectorSubcoreMesh(core_axis_name="c", subcore_axis_name="s")

    @functools.partial(
        pl.kernel,
        mesh=mesh,
        compiler_params=pltpu.CompilerParams(
            use_tc_tiling_on_sc=False, needs_layout_passes=False
        ),
        out_type=jax.ShapeDtypeStruct((H, DT, B // _G, DS, _G), jnp.float32),
        scratch_types=[
            pltpu.VMEM((_XB, _G), jnp.int32),
            pltpu.VMEM((2, _G), jnp.int32),
            pltpu.VMEM((2, _G, D), jnp.float32),
            pltpu.VMEM((2, DT, DS, _G + 1), jnp.float32),
            pltpu.SemaphoreType.DMA,
            pltpu.SemaphoreType.DMA,
            pltpu.SemaphoreType.DMA,
            pltpu.SemaphoreType.DMA,
        ],
    )
    def k(xg_hbm, table_hbm, out_hbm, xbuf, idxbuf, rowbuf, tbuf, g0, g1, w0, w1):
        nc = lax.axis_size("c")
        wid = lax.axis_index("s") * nc + lax.axis_index("c")
        gbase = wid * gpw
        gsem = (g0, g1)
        wsem = (w0, w1)
        lane = jax.lax.iota(jnp.int32, _L)
        dsv = lax.rem(lane, 8)
        dtv = [dv * 2 + lax.div(lane, 8) for dv in range(D // _L)]

        def make_idx(slot, r):
            for v in range(_G // _L):
                s = pl.ds(v * _L, _L)
                idxbuf[slot, s] = -xbuf[r, s] - 1

        def fire_gather(slot):
            pltpu.async_copy(
                table_hbm.at[idxbuf.at[slot]], rowbuf.at[slot], gsem[slot]
            )

        def wait_gather(slot):
            pltpu.make_async_copy(
                table_hbm.at[idxbuf.at[slot]], rowbuf.at[slot], gsem[slot]
            ).wait()

        def transpose(slot):
            def tb(bo, carry):
                for bi in range(_L):
                    b = bo * _L + bi
                    blv = jnp.full((_L,), b, dtype=jnp.int32)
                    for dv in range(D // _L):
                        vec = rowbuf[slot, b, pl.ds(dv * _L, _L)]
                        plsc.store_scatter(
                            tbuf.at[slot], [dtv[dv], dsv, blv], vec
                        )
                return carry

            lax.fori_loop(0, _G // _L, tb, 0, unroll=2)

        def write_desc(slot, g):
            h = lax.div(g, B // _G)
            bt = lax.rem(g, B // _G)
            return pltpu.make_async_copy(
                tbuf.at[slot, :, :, pl.ds(0, _G)], out_hbm.at[h, :, bt],
                wsem[slot],
            )

        def body(p, carry):
            li0 = 2 * p
            gg0 = gbase + li0

            @pl.when(lax.rem(li0, _XB) == 0)
            def _load_x():
                pltpu.sync_copy(xg_hbm.at[pl.ds(gg0, _XB)], xbuf)

            r0 = lax.rem(li0, _XB)
            make_idx(0, r0)
            fire_gather(0)

            @pl.when(p >= 1)
            def _finish_prev_slot1():
                wait_gather(1)

                @pl.when(p >= 2)
                def _w1():
                    write_desc(1, gg0 - 3).wait()

                transpose(1)
                write_desc(1, gg0 - 1).start()

            make_idx(1, r0 + 1)
            fire_gather(1)

            wait_gather(0)

            @pl.when(p >= 1)
            def _w0():
                write_desc(0, gg0 - 2).wait()

            transpose(0)
            write_desc(0, gg0).start()
            return carry

        lax.fori_loop(0, n_pairs, body, 0)
        glast = gbase + gpw - 1
        wait_gather(1)
        write_desc(1, glast - 2).wait()
        transpose(1)
        write_desc(1, glast).start()
        write_desc(0, glast - 1).wait()
        write_desc(1, glast).wait()

    return k


def kernel(x, table):
    B, H = x.shape
    V, D = table.shape
    info = plsc.get_sparse_core_info()
    NW = info.num_cores * info.num_subcores
    xg = x.T.reshape((B // _G) * H, _G).astype(jnp.int32)
    out5 = _build(B, H, V, D, NW)(xg, table)
    return jnp.transpose(out5, (2, 4, 0, 1, 3)).reshape(B, H, D)


# speedup vs baseline: 1.2845x; 1.2845x over previous
import functools

import jax
import jax.numpy as jnp
from jax import lax
from jax.experimental import pallas as pl
from jax.experimental.pallas import tpu as pltpu
from jax.experimental.pallas import tpu_sc as plsc

_L = 16
_G = 128
_XB = 32


def _build(B, H, V, D, NW):
    n_groups = (B // _G) * H
    gpw = n_groups // NW
    n_pairs = gpw // 2
    DT, DS = D // 8, 8

    mesh = plsc.VectorSubcoreMesh(core_axis_name="c", subcore_axis_name="s")

    @functools.partial(
        pl.kernel,
        mesh=mesh,
        compiler_params=pltpu.CompilerParams(
            use_tc_tiling_on_sc=False, needs_layout_passes=False
        ),
        out_type=jax.ShapeDtypeStruct((H, DT, B // _G, DS, _G), jnp.float32),
        scratch_types=[
            pltpu.VMEM((_XB, _G), jnp.int32),
            pltpu.VMEM((2, _G), jnp.int32),
            pltpu.VMEM((2, _G, D), jnp.float32),
            pltpu.VMEM((2, DT, DS, _G + 1), jnp.float32),
            pltpu.SemaphoreType.DMA,
            pltpu.SemaphoreType.DMA,
            pltpu.SemaphoreType.DMA,
            pltpu.SemaphoreType.DMA,
        ],
    )
    def k(xg_hbm, table_hbm, out_hbm, xbuf, idxbuf, rowbuf, tbuf, g0, g1, w0, w1):
        nc = lax.axis_size("c")
        wid = lax.axis_index("s") * nc + lax.axis_index("c")
        gbase = wid * gpw
        gsem = (g0, g1)
        wsem = (w0, w1)
        lane = jax.lax.iota(jnp.int32, _L)
        dsv = lax.rem(lane, 8)
        dtv = [dv * 2 + lax.div(lane, 8) for dv in range(D // _L)]

        def make_idx(slot, r):
            for v in range(_G // _L):
                s = pl.ds(v * _L, _L)
                idxbuf[slot, s] = -xbuf[r, s] - 1

        def fire_gather(slot):
            pltpu.async_copy(
                table_hbm.at[idxbuf.at[slot]], rowbuf.at[slot], gsem[slot]
            )

        def wait_gather(slot):
            pltpu.make_async_copy(
                table_hbm.at[idxbuf.at[slot]], rowbuf.at[slot], gsem[slot]
            ).wait()

        vone = jnp.full((_L,), 1, dtype=jnp.int32)

        def transpose(slot):
            def tb(bo, blv):
                b0 = 2 * bo
                blv1 = blv + vone
                vecs = [
                    rowbuf[slot, b0 + i, pl.ds(dv * _L, _L)]
                    for i in (0, 1)
                    for dv in range(D // _L)
                ]
                for i, bl in ((0, blv), (1, blv1)):
                    for dv in range(D // _L):
                        plsc.store_scatter(
                            tbuf.at[slot],
                            [dtv[dv], dsv, bl],
                            vecs[i * (D // _L) + dv],
                        )
                return blv1 + vone

            lax.fori_loop(
                0, _G // 2, tb, jnp.zeros((_L,), jnp.int32), unroll=2
            )

        def write_desc(slot, g):
            h = lax.div(g, B // _G)
            bt = lax.rem(g, B // _G)
            return pltpu.make_async_copy(
                tbuf.at[slot, :, :, pl.ds(0, _G)], out_hbm.at[h, :, bt],
                wsem[slot],
            )

        def body(p, carry):
            li0 = 2 * p
            gg0 = gbase + li0

            @pl.when(lax.rem(li0, _XB) == 0)
            def _load_x():
                pltpu.sync_copy(xg_hbm.at[pl.ds(gg0, _XB)], xbuf)

            r0 = lax.rem(li0, _XB)
            make_idx(0, r0)
            fire_gather(0)

            @pl.when(p >= 1)
            def _finish_prev_slot1():
                wait_gather(1)

                @pl.when(p >= 2)
                def _w1():
                    write_desc(1, gg0 - 3).wait()

                transpose(1)
                write_desc(1, gg0 - 1).start()

            make_idx(1, r0 + 1)
            fire_gather(1)

            wait_gather(0)

            @pl.when(p >= 1)
            def _w0():
                write_desc(0, gg0 - 2).wait()

            transpose(0)
            write_desc(0, gg0).start()
            return carry

        lax.fori_loop(0, n_pairs, body, 0)
        glast = gbase + gpw - 1
        wait_gather(1)
        write_desc(1, glast - 2).wait()
        transpose(1)
        write_desc(1, glast).start()
        write_desc(0, glast - 1).wait()
        write_desc(1, glast).wait()

    return k


def kernel(x, table):
    B, H = x.shape
    V, D = table.shape
    info = plsc.get_sparse_core_info()
    NW = info.num_cores * info.num_subcores
    xg = x.T.reshape((B // _G) * H, _G).astype(jnp.int32)
    out5 = _build(B, H, V, D, NW)(xg, table)
    return jnp.transpose(out5, (2, 4, 0, 1, 3)).reshape(B, H, D)


# speedup vs baseline: 1.2994x; 1.0116x over previous
import functools

import jax
import jax.numpy as jnp
from jax import lax
from jax.experimental import pallas as pl
from jax.experimental.pallas import tpu as pltpu
from jax.experimental.pallas import tpu_sc as plsc

_L = 16
_G = 128
_XB = 32


def _build(B, H, V, D, NW):
    n_groups = (B // _G) * H
    gpw = n_groups // NW
    n_pairs = gpw // 2
    DT, DS = D // 8, 8

    mesh = plsc.VectorSubcoreMesh(core_axis_name="c", subcore_axis_name="s")

    @functools.partial(
        pl.kernel,
        mesh=mesh,
        compiler_params=pltpu.CompilerParams(
            use_tc_tiling_on_sc=False, needs_layout_passes=False
        ),
        out_type=jax.ShapeDtypeStruct((H, DT, B // _G, DS, _G), jnp.float32),
        scratch_types=[
            pltpu.VMEM((_XB, _G), jnp.int32),
            pltpu.VMEM((2, _G), jnp.int32),
            pltpu.VMEM((2, _G, D), jnp.float32),
            pltpu.VMEM((2, DT, DS, _G + 1), jnp.float32),
            pltpu.SemaphoreType.DMA,
            pltpu.SemaphoreType.DMA,
            pltpu.SemaphoreType.DMA,
            pltpu.SemaphoreType.DMA,
        ],
    )
    def k(xg_hbm, table_hbm, out_hbm, xbuf, idxbuf, rowbuf, tbuf, g0, g1, w0, w1):
        nc = lax.axis_size("c")
        wid = lax.axis_index("s") * nc + lax.axis_index("c")
        gbase = wid * gpw
        gsem = (g0, g1)
        wsem = (w0, w1)
        lane = jax.lax.iota(jnp.int32, _L)
        dsv = lax.rem(lane, 8)
        dtv = [dv * 2 + lax.div(lane, 8) for dv in range(D // _L)]

        def make_idx(slot, r):
            vs = [xbuf[r, pl.ds(v * _L, _L)] for v in range(_G // _L)]
            for v in range(_G // _L):
                idxbuf[slot, pl.ds(v * _L, _L)] = -vs[v] - 1

        def fire_gather(slot):
            pltpu.async_copy(
                table_hbm.at[idxbuf.at[slot]], rowbuf.at[slot], gsem[slot]
            )

        def wait_gather(slot):
            pltpu.make_async_copy(
                table_hbm.at[idxbuf.at[slot]], rowbuf.at[slot], gsem[slot]
            ).wait()

        vone = jnp.full((_L,), 1, dtype=jnp.int32)

        def transpose(slot):
            def tb(bo, blv):
                b0 = 2 * bo
                blv1 = blv + vone
                vecs = [
                    rowbuf[slot, b0 + i, pl.ds(dv * _L, _L)]
                    for i in (0, 1)
                    for dv in range(D // _L)
                ]
                for i, bl in ((0, blv), (1, blv1)):
                    for dv in range(D // _L):
                        plsc.store_scatter(
                            tbuf.at[slot],
                            [dtv[dv], dsv, bl],
                            vecs[i * (D // _L) + dv],
                        )
                return blv1 + vone

            lax.fori_loop(
                0, _G // 2, tb, jnp.zeros((_L,), jnp.int32), unroll=2
            )

        def write_desc(slot, g):
            h = lax.div(g, B // _G)
            bt = lax.rem(g, B // _G)
            return pltpu.make_async_copy(
                tbuf.at[slot, :, :, pl.ds(0, _G)], out_hbm.at[h, :, bt],
                wsem[slot],
            )

        def body(p, carry):
            li0 = 2 * p
            gg0 = gbase + li0

            @pl.when(lax.rem(li0, _XB) == 0)
            def _load_x():
                ht = lax.div(gg0, (B // _G) * 8)
                hi = lax.rem(lax.div(gg0, B // _G), 8)
                bt0 = lax.rem(gg0, B // _G)
                pltpu.sync_copy(
                    xg_hbm.at[ht, pl.ds(bt0, _XB), hi], xbuf
                )

            r0 = lax.rem(li0, _XB)
            make_idx(0, r0)
            fire_gather(0)

            @pl.when(p >= 1)
            def _finish_prev_slot1():
                wait_gather(1)

                @pl.when(p >= 2)
                def _w1():
                    write_desc(1, gg0 - 3).wait()

                transpose(1)
                write_desc(1, gg0 - 1).start()

            make_idx(1, r0 + 1)
            fire_gather(1)

            wait_gather(0)

            @pl.when(p >= 1)
            def _w0():
                write_desc(0, gg0 - 2).wait()

            transpose(0)
            write_desc(0, gg0).start()
            return carry

        lax.fori_loop(0, n_pairs, body, 0)
        glast = gbase + gpw - 1
        wait_gather(1)
        write_desc(1, glast - 2).wait()
        transpose(1)
        write_desc(1, glast).start()
        write_desc(0, glast - 1).wait()
        write_desc(1, glast).wait()

    return k


def kernel(x, table):
    B, H = x.shape
    V, D = table.shape
    info = plsc.get_sparse_core_info()
    NW = info.num_cores * info.num_subcores
    xg = (
        x.astype(jnp.int32)
        .T.reshape(H // 8, 8, B // _G, _G)
        .transpose(0, 2, 1, 3)
    )
    out5 = _build(B, H, V, D, NW)(xg, table)
    return jnp.transpose(out5, (2, 4, 0, 1, 3)).reshape(B, H, D)
